# baseline (speedup 1.0000x reference)
import functools

import jax
import jax.numpy as jnp
from jax import lax
from jax.experimental import pallas as pl
from jax.experimental.pallas import tpu as pltpu
from jax.experimental.pallas import tpu_sc as plsc

_L = 1024
_NT = 16
_RPT = _L // _NT
_CH = _L // 16
_NEG = -1e30
_BIGK = 1 << 22


def _sc_body(con_hbm, conT_hbm, seq_hbm, out_hbm, w_v, stage_v, seq_v,
             pairs_v, free_v, bv_v, bk_v, rec_v, mval_v, mcol_v,
             bv_sh, bk_sh, rec_sh):
    cid = lax.axis_index("c")
    sid = lax.axis_index("s")

    @pl.when(cid == 0)
    def _():
        base = sid * _RPT

        pltpu.sync_copy(seq_hbm, seq_v)

        def pairs_body(c, carry):
            sl = pl.ds(c * 16, 16)
            s0 = seq_v[0, sl]
            s1 = seq_v[1, sl]
            s2 = seq_v[2, sl]
            s3 = seq_v[3, sl]
            m = jnp.maximum(jnp.maximum(s0, s1), jnp.maximum(s2, s3))
            p = jnp.where(
                s0 == m, 2.0, jnp.where(s1 == m, 3.0, jnp.where(s2 == m, 5.0, 7.0))
            ).astype(jnp.float32)
            pairs_v[sl] = p
            free_v[sl] = jnp.full((16,), 1.0, jnp.float32)
            return carry

        lax.fori_loop(0, _CH, pairs_body, 0)

        def initm(g, carry):
            sl = pl.ds(g * 16, 16)
            mval_v[sl] = jnp.zeros((16,), jnp.float32)
            mcol_v[sl] = jnp.full((16,), -1, jnp.int32)
            return carry

        lax.fori_loop(0, _RPT // 16, initm, 0)

        pltpu.sync_copy(con_hbm.at[pl.ds(base, _RPT)], w_v)

        def build_group(g, carry):
            pltpu.sync_copy(conT_hbm.at[pl.ds(base + g * 16, 16)], stage_v)

            def row_body(r, carry2):
                row = g * 16 + r
                i = base + row
                pair_i = plsc.load_gather(
                    pairs_v, [jnp.full((16,), i, jnp.int32)]
                )

                def chunk_body(c, carry3):
                    sl = pl.ds(c * 16, 16)
                    col = c * 16 + lax.iota(jnp.int32, 16)
                    v = jnp.where(col - i >= 5, w_v[row, sl], stage_v[r, sl])
                    band = jnp.abs(col - i) >= 5
                    prod = pair_i * pairs_v[sl]
                    ok = (prod == 14.0) | (prod == 15.0) | (prod == 35.0)
                    w_v[row, sl] = jnp.where(band & ok, v, 0.0)
                    return carry3

                lax.fori_loop(0, _CH, chunk_body, 0)
                return carry2

            lax.fori_loop(0, 16, row_body, 0)
            return carry

        lax.fori_loop(0, _RPT // 16, build_group, 0)

        def round_cond(gmax):
            return gmax > 0.0

        def round_body(gmax):
            def row_group(g, carry):
                def one_row(r, st):
                    bv16, bk16 = st
                    row = g * 16 + r
                    i = base + row
                    rowfree = jnp.max(
                        plsc.load_gather(free_v, [jnp.full((16,), i, jnp.int32)])
                    )

                    def do_scan():
                        def chunk(c, st2):
                            maxv, key = st2
                            sl = pl.ds(c * 16, 16)
                            col = c * 16 + lax.iota(jnp.int32, 16)
                            v = jnp.where(free_v[sl] > 0.0, w_v[row, sl], _NEG)
                            kvec = jnp.where(
                                col < i, col * 1024 + i, i * 1024 + col
                            )
                            gt = v > maxv
                            eq = v == maxv
                            nmax = jnp.maximum(v, maxv)
                            nkey = jnp.where(
                                gt, kvec, jnp.where(eq, jnp.minimum(key, kvec), key)
                            )
                            return nmax, nkey

                        maxv, key = lax.fori_loop(
                            0,
                            _CH,
                            chunk,
                            (
                                jnp.full((16,), _NEG, jnp.float32),
                                jnp.full((16,), _BIGK, jnp.int32),
                            ),
                        )
                        bv = jnp.max(maxv)
                        bk = jnp.min(jnp.where(maxv == bv, key, _BIGK))
                        return bv, bk

                    def no_scan():
                        return jnp.float32(_NEG), jnp.int32(_BIGK)

                    bv, bk = lax.cond(rowfree > 0.0, do_scan, no_scan)
                    lane = lax.iota(jnp.int32, 16) == r
                    return jnp.where(lane, bv, bv16), jnp.where(lane, bk, bk16)

                bv16, bk16 = lax.fori_loop(
                    0,
                    16,
                    one_row,
                    (
                        jnp.full((16,), _NEG, jnp.float32),
                        jnp.full((16,), _BIGK, jnp.int32),
                    ),
                )
                sl = pl.ds(base + g * 16, 16)
                bv_v[sl] = bv16
                bk_v[sl] = bk16
                return carry

            lax.fori_loop(0, _RPT // 16, row_group, 0)

            pltpu.sync_copy(bv_v.at[pl.ds(base, _RPT)], bv_sh.at[pl.ds(base, _RPT)])
            pltpu.sync_copy(bk_v.at[pl.ds(base, _RPT)], bk_sh.at[pl.ds(base, _RPT)])
            plsc.subcore_barrier()
            pltpu.sync_copy(bv_sh, bv_v)
            pltpu.sync_copy(bk_sh, bk_v)

            def rec_group(g, carry):
                sl_g = pl.ds(base + g * 16, 16)
                i_vec = base + g * 16 + lax.iota(jnp.int32, 16)
                bv = bv_v[sl_g]
                bk = bk_v[sl_g]
                m1 = lax.shift_right_arithmetic(bk, 10)
                m2 = jnp.bitwise_and(bk, 1023)
                j = jnp.where(m1 == i_vec, m2, m1)
                jc = jnp.clip(j, 0, _L - 1)
                bk_j = plsc.load_gather(bk_v, [jc])
                rec = (bv > 0.0) & (bk_j == bk)
                msl = pl.ds(g * 16, 16)
                mval_v[msl] = jnp.where(rec, bv, mval_v[msl])
                mcol_v[msl] = jnp.where(rec, jc, mcol_v[msl])
                rec_v[sl_g] = jnp.where(rec, 1.0, 0.0)
                return carry

            lax.fori_loop(0, _RPT // 16, rec_group, 0)

            pltpu.sync_copy(rec_v.at[pl.ds(base, _RPT)], rec_sh.at[pl.ds(base, _RPT)])
            plsc.subcore_barrier()
            pltpu.sync_copy(rec_sh, rec_v)

            def upd(c, m):
                sl = pl.ds(c * 16, 16)
                free_v[sl] = free_v[sl] * (1.0 - rec_v[sl])
                return jnp.maximum(m, jnp.max(bv_v[sl]))

            gmax2 = lax.fori_loop(0, _CH, upd, jnp.float32(_NEG))
            return gmax2

        lax.while_loop(round_cond, round_body, jnp.float32(1.0))

        def zero_row(row, carry):
            def zc(c, carry2):
                w_v[row, pl.ds(c * 16, 16)] = jnp.zeros((16,), jnp.float32)
                return carry2

            lax.fori_loop(0, _CH, zc, 0)
            return carry

        lax.fori_loop(0, _RPT, zero_row, 0)

        def out_group(g, carry):
            msl = pl.ds(g * 16, 16)
            mc = mcol_v[msl]
            mv = mval_v[msl]
            rowidx = g * 16 + lax.iota(jnp.int32, 16)
            plsc.store_scatter(
                w_v, [rowidx, jnp.clip(mc, 0, _L - 1)], mv, mask=mc >= 0
            )
            return carry

        lax.fori_loop(0, _RPT // 16, out_group, 0)
        pltpu.sync_copy(w_v, out_hbm.at[pl.ds(base, _RPT)])


_sc_call = functools.partial(
    pl.kernel,
    out_type=jax.ShapeDtypeStruct((_L, _L), jnp.float32),
    mesh=plsc.VectorSubcoreMesh(core_axis_name="c", subcore_axis_name="s"),
    compiler_params=pltpu.CompilerParams(needs_layout_passes=False),
    scratch_types=[
        pltpu.VMEM((_RPT, _L), jnp.float32),
        pltpu.VMEM((16, _L), jnp.float32),
        pltpu.VMEM((4, _L), jnp.float32),
        pltpu.VMEM((_L,), jnp.float32),
        pltpu.VMEM((_L,), jnp.float32),
        pltpu.VMEM((_L,), jnp.float32),
        pltpu.VMEM((_L,), jnp.int32),
        pltpu.VMEM((_L,), jnp.float32),
        pltpu.VMEM((_RPT,), jnp.float32),
        pltpu.VMEM((_RPT,), jnp.int32),
        pltpu.VMEM_SHARED((_L,), jnp.float32),
        pltpu.VMEM_SHARED((_L,), jnp.int32),
        pltpu.VMEM_SHARED((_L,), jnp.float32),
    ],
)


def kernel(con, feat):
    con2d = con.reshape(_L, _L)
    conT = jnp.swapaxes(con2d, 0, 1)
    seq = feat[0, :4, :, 0]
    out = _sc_call(_sc_body)(con2d, conT, seq)
    return out.reshape(con.shape)


# speedup vs baseline: 1.7256x; 1.7256x over previous
import functools

import jax
import jax.numpy as jnp
from jax import lax
from jax.experimental import pallas as pl
from jax.experimental.pallas import tpu as pltpu
from jax.experimental.pallas import tpu_sc as plsc

_L = 1024
_NT = 16
_RPT = _L // _NT
_CH = _L // 16
_NEG = -1e30


def _sc_body(con_hbm, conT_hbm, seq_hbm, out_hbm, w_v, stage_v, seq_v,
             pairs_v, free_v, bv_v, bc_v, rec_v, mval_v, mcol_v,
             bv_sh, bc_sh, rec_sh):
    cid = lax.axis_index("c")
    sid = lax.axis_index("s")

    @pl.when(cid == 0)
    def _():
        base = sid * _RPT

        pltpu.sync_copy(seq_hbm, seq_v)

        @plsc.parallel_loop(0, _CH, unroll=4)
        def _pairs(c):
            sl = pl.ds(c * 16, 16)
            s0 = seq_v[0, sl]
            s1 = seq_v[1, sl]
            s2 = seq_v[2, sl]
            s3 = seq_v[3, sl]
            m = jnp.maximum(jnp.maximum(s0, s1), jnp.maximum(s2, s3))
            p = jnp.where(
                s0 == m, 2.0, jnp.where(s1 == m, 3.0, jnp.where(s2 == m, 5.0, 7.0))
            ).astype(jnp.float32)
            pairs_v[sl] = p
            free_v[sl] = jnp.full((16,), 1.0, jnp.float32)

        @plsc.parallel_loop(0, _RPT // 16, unroll=1)
        def _initm(g):
            sl = pl.ds(g * 16, 16)
            mval_v[sl] = jnp.zeros((16,), jnp.float32)
            mcol_v[sl] = jnp.full((16,), -1, jnp.int32)

        pltpu.sync_copy(con_hbm.at[pl.ds(base, _RPT)], w_v)

        def build_group(g, carry):
            pltpu.sync_copy(conT_hbm.at[pl.ds(base + g * 16, 16)], stage_v)

            def row_body(r, carry2):
                row = g * 16 + r
                i = base + row
                pair_i = plsc.load_gather(
                    pairs_v, [jnp.full((16,), i, jnp.int32)]
                )

                @plsc.parallel_loop(0, _CH, unroll=4)
                def _chunk(c):
                    sl = pl.ds(c * 16, 16)
                    col = c * 16 + lax.iota(jnp.int32, 16)
                    v = jnp.where(col - i >= 5, w_v[row, sl], stage_v[r, sl])
                    band = jnp.abs(col - i) >= 5
                    prod = pair_i * pairs_v[sl]
                    ok = (prod == 14.0) | (prod == 15.0) | (prod == 35.0)
                    w_v[row, sl] = jnp.where(band & ok, v, 0.0)

                return carry2

            lax.fori_loop(0, 16, row_body, 0)
            return carry

        lax.fori_loop(0, _RPT // 16, build_group, 0)

        def round_cond(gmax):
            return gmax > 0.0

        def round_body(gmax):
            def row_group(g, carry):
                def one_row(r, st):
                    bv16, bc16 = st
                    row = g * 16 + r
                    i = base + row
                    rowfree = jnp.max(
                        plsc.load_gather(free_v, [jnp.full((16,), i, jnp.int32)])
                    )

                    def do_scan():
                        init = (
                            jnp.full((16,), _NEG, jnp.float32),
                            jnp.full((16,), _L, jnp.int32),
                        )

                        @plsc.parallel_loop(0, _CH, unroll=4, carry=init)
                        def scan(c, st2):
                            maxv, bcol = st2
                            sl = pl.ds(c * 16, 16)
                            col = c * 16 + lax.iota(jnp.int32, 16)
                            v = jnp.where(free_v[sl] > 0.0, w_v[row, sl], _NEG)
                            gt = v > maxv
                            return (
                                jnp.maximum(v, maxv),
                                jnp.where(gt, col, bcol),
                            )

                        maxv, bcol = scan
                        bv = jnp.max(maxv)
                        bc = jnp.min(jnp.where(maxv == bv, bcol, _L))
                        return bv, bc

                    def no_scan():
                        return jnp.float32(_NEG), jnp.int32(-1)

                    bv, bc = lax.cond(rowfree > 0.0, do_scan, no_scan)
                    lane = lax.iota(jnp.int32, 16) == r
                    return jnp.where(lane, bv, bv16), jnp.where(lane, bc, bc16)

                bv16, bc16 = lax.fori_loop(
                    0,
                    16,
                    one_row,
                    (
                        jnp.full((16,), _NEG, jnp.float32),
                        jnp.full((16,), -1, jnp.int32),
                    ),
                )
                sl = pl.ds(base + g * 16, 16)
                bv_v[sl] = bv16
                bc_v[sl] = bc16
                return carry

            lax.fori_loop(0, _RPT // 16, row_group, 0)

            pltpu.sync_copy(bv_v.at[pl.ds(base, _RPT)], bv_sh.at[pl.ds(base, _RPT)])
            pltpu.sync_copy(bc_v.at[pl.ds(base, _RPT)], bc_sh.at[pl.ds(base, _RPT)])
            plsc.subcore_barrier()
            pltpu.sync_copy(bv_sh, bv_v)
            pltpu.sync_copy(bc_sh, bc_v)

            def rec_group(g, carry):
                sl_g = pl.ds(base + g * 16, 16)
                i_vec = base + g * 16 + lax.iota(jnp.int32, 16)
                bv = bv_v[sl_g]
                bc = bc_v[sl_g]
                jc = jnp.clip(bc, 0, _L - 1)
                bc_j = plsc.load_gather(bc_v, [jc])
                rec = (bv > 0.0) & (bc_j == i_vec)
                msl = pl.ds(g * 16, 16)
                mval_v[msl] = jnp.where(rec, bv, mval_v[msl])
                mcol_v[msl] = jnp.where(rec, jc, mcol_v[msl])
                rec_v[sl_g] = jnp.where(rec, 1.0, 0.0)
                return carry

            lax.fori_loop(0, _RPT // 16, rec_group, 0)

            pltpu.sync_copy(rec_v.at[pl.ds(base, _RPT)], rec_sh.at[pl.ds(base, _RPT)])
            plsc.subcore_barrier()
            pltpu.sync_copy(rec_sh, rec_v)

            @plsc.parallel_loop(0, _CH, unroll=4, carry=jnp.float32(_NEG))
            def upd(c, m):
                sl = pl.ds(c * 16, 16)
                free_v[sl] = free_v[sl] * (1.0 - rec_v[sl])
                return jnp.maximum(m, jnp.max(bv_v[sl]))

            return upd

        lax.while_loop(round_cond, round_body, jnp.float32(1.0))

        def zero_row(row, carry):
            @plsc.parallel_loop(0, _CH, unroll=4)
            def _zc(c):
                w_v[row, pl.ds(c * 16, 16)] = jnp.zeros((16,), jnp.float32)

            return carry

        lax.fori_loop(0, _RPT, zero_row, 0)

        def out_group(g, carry):
            msl = pl.ds(g * 16, 16)
            mc = mcol_v[msl]
            mv = mval_v[msl]
            rowidx = g * 16 + lax.iota(jnp.int32, 16)
            plsc.store_scatter(
                w_v, [rowidx, jnp.clip(mc, 0, _L - 1)], mv, mask=mc >= 0
            )
            return carry

        lax.fori_loop(0, _RPT // 16, out_group, 0)
        pltpu.sync_copy(w_v, out_hbm.at[pl.ds(base, _RPT)])


_sc_call = functools.partial(
    pl.kernel,
    out_type=jax.ShapeDtypeStruct((_L, _L), jnp.float32),
    mesh=plsc.VectorSubcoreMesh(core_axis_name="c", subcore_axis_name="s"),
    compiler_params=pltpu.CompilerParams(needs_layout_passes=False),
    scratch_types=[
        pltpu.VMEM((_RPT, _L), jnp.float32),
        pltpu.VMEM((16, _L), jnp.float32),
        pltpu.VMEM((4, _L), jnp.float32),
        pltpu.VMEM((_L,), jnp.float32),
        pltpu.VMEM((_L,), jnp.float32),
        pltpu.VMEM((_L,), jnp.float32),
        pltpu.VMEM((_L,), jnp.int32),
        pltpu.VMEM((_L,), jnp.float32),
        pltpu.VMEM((_RPT,), jnp.float32),
        pltpu.VMEM((_RPT,), jnp.int32),
        pltpu.VMEM_SHARED((_L,), jnp.float32),
        pltpu.VMEM_SHARED((_L,), jnp.int32),
        pltpu.VMEM_SHARED((_L,), jnp.float32),
    ],
)


def kernel(con, feat):
    con2d = con.reshape(_L, _L)
    conT = jnp.swapaxes(con2d, 0, 1)
    seq = feat[0, :4, :, 0]
    out = _sc_call(_sc_body)(con2d, conT, seq)
    return out.reshape(con.shape)


# speedup vs baseline: 1.7823x; 1.0329x over previous
import functools

import jax
import jax.numpy as jnp
from jax import lax
from jax.experimental import pallas as pl
from jax.experimental.pallas import tpu as pltpu
from jax.experimental.pallas import tpu_sc as plsc

_L = 1024
_NT = 16
_RPT = _L // _NT
_CH = _L // 16
_NEG = -1e30


def _sc_body(con_hbm, conT_hbm, seq_hbm, out_hbm, w_v, stage_v, seq_v,
             pairs_v, free_v, bv_v, bc_v, mval_v, mcol_v,
             bv_sh, bc_sh):
    cid = lax.axis_index("c")
    sid = lax.axis_index("s")

    @pl.when(cid == 0)
    def _():
        base = sid * _RPT

        pltpu.sync_copy(seq_hbm, seq_v)

        @plsc.parallel_loop(0, _CH, unroll=4)
        def _pairs(c):
            sl = pl.ds(c * 16, 16)
            s0 = seq_v[0, sl]
            s1 = seq_v[1, sl]
            s2 = seq_v[2, sl]
            s3 = seq_v[3, sl]
            m = jnp.maximum(jnp.maximum(s0, s1), jnp.maximum(s2, s3))
            p = jnp.where(
                s0 == m, 2.0, jnp.where(s1 == m, 3.0, jnp.where(s2 == m, 5.0, 7.0))
            ).astype(jnp.float32)
            pairs_v[sl] = p
            free_v[sl] = jnp.full((16,), 1.0, jnp.float32)

        @plsc.parallel_loop(0, _RPT // 16, unroll=1)
        def _initm(g):
            sl = pl.ds(g * 16, 16)
            mval_v[sl] = jnp.zeros((16,), jnp.float32)
            mcol_v[sl] = jnp.full((16,), -1, jnp.int32)

        pltpu.sync_copy(con_hbm.at[pl.ds(base, _RPT)], w_v)

        def build_group(g, carry):
            pltpu.sync_copy(conT_hbm.at[pl.ds(base + g * 16, 16)], stage_v)

            def row_body(r, carry2):
                row = g * 16 + r
                i = base + row
                pair_i = plsc.load_gather(
                    pairs_v, [jnp.full((16,), i, jnp.int32)]
                )

                @plsc.parallel_loop(0, _CH, unroll=4)
                def _chunk(c):
                    sl = pl.ds(c * 16, 16)
                    col = c * 16 + lax.iota(jnp.int32, 16)
                    v = jnp.where(col - i >= 5, w_v[row, sl], stage_v[r, sl])
                    band = jnp.abs(col - i) >= 5
                    prod = pair_i * pairs_v[sl]
                    ok = (prod == 14.0) | (prod == 15.0) | (prod == 35.0)
                    w_v[row, sl] = jnp.where(band & ok, v, 0.0)

                return carry2

            lax.fori_loop(0, 16, row_body, 0)
            return carry

        lax.fori_loop(0, _RPT // 16, build_group, 0)

        def round_cond(gmax):
            return gmax > 0.0

        def round_body(gmax):
            def row_group(g, carry):
                def one_row(r, st):
                    bv16, bc16 = st
                    row = g * 16 + r
                    i = base + row
                    rowfree = jnp.max(
                        plsc.load_gather(free_v, [jnp.full((16,), i, jnp.int32)])
                    )

                    def do_scan():
                        init = (
                            jnp.full((16,), _NEG, jnp.float32),
                            jnp.full((16,), _L, jnp.int32),
                        )

                        @plsc.parallel_loop(0, _CH, unroll=8, carry=init)
                        def scan(c, st2):
                            maxv, bcol = st2
                            sl = pl.ds(c * 16, 16)
                            col = c * 16 + lax.iota(jnp.int32, 16)
                            v = jnp.where(free_v[sl] > 0.0, w_v[row, sl], _NEG)
                            gt = v > maxv
                            return (
                                jnp.maximum(v, maxv),
                                jnp.where(gt, col, bcol),
                            )

                        maxv, bcol = scan
                        bv = jnp.max(maxv)
                        bc = jnp.min(jnp.where(maxv == bv, bcol, _L))
                        return bv, bc

                    def no_scan():
                        return jnp.float32(_NEG), jnp.int32(-1)

                    bv, bc = lax.cond(rowfree > 0.0, do_scan, no_scan)
                    lane = lax.iota(jnp.int32, 16) == r
                    return jnp.where(lane, bv, bv16), jnp.where(lane, bc, bc16)

                bv16, bc16 = lax.fori_loop(
                    0,
                    16,
                    one_row,
                    (
                        jnp.full((16,), _NEG, jnp.float32),
                        jnp.full((16,), -1, jnp.int32),
                    ),
                )
                sl = pl.ds(base + g * 16, 16)
                bv_v[sl] = bv16
                bc_v[sl] = bc16
                return carry

            lax.fori_loop(0, _RPT // 16, row_group, 0)

            pltpu.sync_copy(bv_v.at[pl.ds(base, _RPT)], bv_sh.at[pl.ds(base, _RPT)])
            pltpu.sync_copy(bc_v.at[pl.ds(base, _RPT)], bc_sh.at[pl.ds(base, _RPT)])
            plsc.subcore_barrier()
            pltpu.sync_copy(bv_sh, bv_v)
            pltpu.sync_copy(bc_sh, bc_v)

            def rec_group(g, carry):
                sl_g = pl.ds(base + g * 16, 16)
                i_vec = base + g * 16 + lax.iota(jnp.int32, 16)
                bv = bv_v[sl_g]
                bc = bc_v[sl_g]
                jc = jnp.clip(bc, 0, _L - 1)
                bc_j = plsc.load_gather(bc_v, [jc])
                rec = (bv > 0.0) & (bc_j == i_vec)
                msl = pl.ds(g * 16, 16)
                mval_v[msl] = jnp.where(rec, bv, mval_v[msl])
                mcol_v[msl] = jnp.where(rec, jc, mcol_v[msl])
                return carry

            lax.fori_loop(0, _RPT // 16, rec_group, 0)

            @plsc.parallel_loop(0, _CH, unroll=4, carry=jnp.float32(_NEG))
            def upd(c, m):
                sl = pl.ds(c * 16, 16)
                i_vec = c * 16 + lax.iota(jnp.int32, 16)
                bv = bv_v[sl]
                bc = bc_v[sl]
                jc = jnp.clip(bc, 0, _L - 1)
                bc_j = plsc.load_gather(bc_v, [jc])
                rec = (bv > 0.0) & (bc_j == i_vec)
                free_v[sl] = jnp.where(rec, 0.0, free_v[sl])
                return jnp.maximum(m, jnp.max(bv))

            return upd

        lax.while_loop(round_cond, round_body, jnp.float32(1.0))

        def zero_row(row, carry):
            @plsc.parallel_loop(0, _CH, unroll=4)
            def _zc(c):
                w_v[row, pl.ds(c * 16, 16)] = jnp.zeros((16,), jnp.float32)

            return carry

        lax.fori_loop(0, _RPT, zero_row, 0)

        def out_group(g, carry):
            msl = pl.ds(g * 16, 16)
            mc = mcol_v[msl]
            mv = mval_v[msl]
            rowidx = g * 16 + lax.iota(jnp.int32, 16)
            plsc.store_scatter(
                w_v, [rowidx, jnp.clip(mc, 0, _L - 1)], mv, mask=mc >= 0
            )
            return carry

        lax.fori_loop(0, _RPT // 16, out_group, 0)
        pltpu.sync_copy(w_v, out_hbm.at[pl.ds(base, _RPT)])


_sc_call = functools.partial(
    pl.kernel,
    out_type=jax.ShapeDtypeStruct((_L, _L), jnp.float32),
    mesh=plsc.VectorSubcoreMesh(core_axis_name="c", subcore_axis_name="s"),
    compiler_params=pltpu.CompilerParams(needs_layout_passes=False),
    scratch_types=[
        pltpu.VMEM((_RPT, _L), jnp.float32),
        pltpu.VMEM((16, _L), jnp.float32),
        pltpu.VMEM((4, _L), jnp.float32),
        pltpu.VMEM((_L,), jnp.float32),
        pltpu.VMEM((_L,), jnp.float32),
        pltpu.VMEM((_L,), jnp.float32),
        pltpu.VMEM((_L,), jnp.int32),
        pltpu.VMEM((_RPT,), jnp.float32),
        pltpu.VMEM((_RPT,), jnp.int32),
        pltpu.VMEM_SHARED((_L,), jnp.float32),
        pltpu.VMEM_SHARED((_L,), jnp.int32),
    ],
)


def kernel(con, feat):
    con2d = con.reshape(_L, _L)
    conT = jnp.swapaxes(con2d, 0, 1)
    seq = feat[0, :4, :, 0]
    out = _sc_call(_sc_body)(con2d, conT, seq)
    return out.reshape(con.shape)
